# baseline (speedup 1.0000x reference)
import functools

import jax
import jax.numpy as jnp
from jax import lax
from jax.experimental import pallas as pl
from jax.experimental.pallas import tpu as pltpu
from jax.experimental.pallas import tpu_sc as plsc

N = 10000
E = 160000
H = 256
HH = 128
L = 2

NTILES = 16
CHUNK = 128
NCHUNK = 80
EPAD = NTILES * NCHUNK * CHUNK

R1 = 9712
NSP1 = 9728
NSP2 = 1024
OUT1 = 10240
TAILOFF = 9000

BR = 1000
TAILBLK = N // BR - 1
TAILROW = R1 - TAILBLK * BR


def _tc_a_body(h_ref, w_ref, whh_ref, bhh_ref, m2_ref, gh_ref):
    h_blk = h_ref[...]
    m = jnp.dot(h_blk, w_ref[...], preferred_element_type=jnp.float32)
    m2_ref[0] = m[:, :HH]
    m2_ref[1] = m[:, HH:]
    gh = lax.dot_general(h_blk, whh_ref[...], (((1,), (1,)), ((), ())),
                         preferred_element_type=jnp.float32)
    gh_ref[...] = gh + bhh_ref[...]


def _tc_a(h, w, w_hh, b_hh2):
    return pl.pallas_call(
        _tc_a_body,
        grid=(N // BR,),
        in_specs=[
            pl.BlockSpec((BR, H), lambda r: (r, 0)),
            pl.BlockSpec((H, H), lambda r: (0, 0)),
            pl.BlockSpec((3 * H, H), lambda r: (0, 0)),
            pl.BlockSpec((1, 3 * H), lambda r: (0, 0)),
        ],
        out_specs=[
            pl.BlockSpec((2, BR, HH), lambda r: (0, r, 0)),
            pl.BlockSpec((BR, 3 * H), lambda r: (r, 0)),
        ],
        out_shape=[
            jax.ShapeDtypeStruct((2, N, HH), jnp.float32),
            jax.ShapeDtypeStruct((N, 3 * H), jnp.float32),
        ],
    )(h, w, w_hh, b_hh2)


def _tc_b_body(agg1_ref, agg2_ref, h_ref, gh_ref, wih_ref, bih_ref, out_ref):
    r = pl.program_id(0)
    row = lax.broadcasted_iota(jnp.int32, (BR, HH), 0)
    tail = jnp.logical_and(r == TAILBLK, row >= TAILROW)
    lo = jnp.where(tail, agg2_ref[0], agg1_ref[0])
    hi = jnp.where(tail, agg2_ref[1], agg1_ref[1])
    agg = jnp.concatenate([lo, hi], axis=1)
    gi = lax.dot_general(agg, wih_ref[...], (((1,), (1,)), ((), ())),
                         preferred_element_type=jnp.float32)
    gi = gi + bih_ref[...]
    gh = gh_ref[...]
    h_blk = h_ref[...]
    i_r = gi[:, :H]
    i_z = gi[:, H:2 * H]
    i_n = gi[:, 2 * H:]
    h_r = gh[:, :H]
    h_z = gh[:, H:2 * H]
    h_n = gh[:, 2 * H:]
    r = jax.nn.sigmoid(i_r + h_r)
    z = jax.nn.sigmoid(i_z + h_z)
    n = jnp.tanh(i_n + r * h_n)
    out_ref[...] = (1.0 - z) * n + z * h_blk


def _tc_b(agg1, agg2, h, gh, w_ih, b_ih2):
    return pl.pallas_call(
        _tc_b_body,
        grid=(N // BR,),
        in_specs=[
            pl.BlockSpec((2, BR, HH), lambda r: (0, r, 0)),
            pl.BlockSpec((2, BR, HH), lambda r: (0, 0, 0)),
            pl.BlockSpec((BR, H), lambda r: (r, 0)),
            pl.BlockSpec((BR, 3 * H), lambda r: (r, 0)),
            pl.BlockSpec((3 * H, H), lambda r: (0, 0)),
            pl.BlockSpec((1, 3 * H), lambda r: (0, 0)),
        ],
        out_specs=pl.BlockSpec((BR, H), lambda r: (r, 0)),
        out_shape=jax.ShapeDtypeStruct((N, H), jnp.float32),
    )(agg1, agg2, h, gh, w_ih, b_ih2)


def _make_sc(nsp, out_rows):
    zrows = nsp // NTILES
    assert zrows % 8 == 0

    def body(m_hbm, srcx_hbm, dst_hbm, zeros_hbm, out_hbm,
             src_v, dst_v, buf0, buf1, agg_sh, sem0, sem1):
        c = lax.axis_index("c")
        s = lax.axis_index("s")
        pltpu.sync_copy(srcx_hbm.at[c, s], src_v)
        pltpu.sync_copy(dst_hbm.at[s], dst_v)
        pltpu.sync_copy(zeros_hbm.at[pl.ds(0, zrows)],
                        agg_sh.at[pl.ds(s * zrows, zrows)])
        plsc.subcore_barrier()

        pltpu.async_copy(m_hbm.at[src_v.at[0]], buf0, sem0)
        pltpu.async_copy(m_hbm.at[src_v.at[1]], buf1, sem1)

        def loop(j2, carry):
            j = j2 * 2
            pltpu.make_async_copy(m_hbm.at[src_v.at[j]], buf0, sem0).wait()
            pltpu.sync_copy(buf0, agg_sh.at[dst_v.at[j]], add=True)

            @pl.when(j2 < NCHUNK // 2 - 1)
            def _():
                pltpu.async_copy(m_hbm.at[src_v.at[j + 2]], buf0, sem0)

            pltpu.make_async_copy(m_hbm.at[src_v.at[j + 1]], buf1,
                                  sem1).wait()
            pltpu.sync_copy(buf1, agg_sh.at[dst_v.at[j + 1]], add=True)

            @pl.when(j2 < NCHUNK // 2 - 1)
            def _():
                pltpu.async_copy(m_hbm.at[src_v.at[j + 3]], buf1, sem1)

            return carry

        lax.fori_loop(0, NCHUNK // 2, loop, 0)
        plsc.subcore_barrier()
        pltpu.sync_copy(agg_sh.at[pl.ds(s * zrows, zrows)],
                        out_hbm.at[c, pl.ds(s * zrows, zrows)])

    return functools.partial(
        pl.kernel,
        out_type=jax.ShapeDtypeStruct((2, out_rows, HH), jnp.float32),
        mesh=plsc.VectorSubcoreMesh(core_axis_name="c", subcore_axis_name="s"),
        scratch_types=(
            [pltpu.VMEM((NCHUNK, CHUNK), jnp.int32) for _ in range(2)]
            + [pltpu.VMEM((CHUNK, HH), jnp.float32) for _ in range(2)]
            + [pltpu.VMEM_SHARED((nsp, HH), jnp.float32)]
            + [pltpu.SemaphoreType.DMA for _ in range(2)]
        ),
    )(body)


_sc_scatter1 = _make_sc(NSP1, OUT1)
_sc_scatter2 = _make_sc(NSP2, NSP2)


@jax.jit
def kernel(x, edge_index, weight, w_ih, w_hh, b_ih, b_hh):
    src = edge_index[0]
    dst = edge_index[1]
    npad = EPAD - E
    spread = jnp.arange(EPAD, dtype=jnp.int32) % NTILES
    src_p = jnp.concatenate(
        [src, jnp.arange(npad, dtype=jnp.int32) % NTILES])
    dst_p = jnp.concatenate(
        [dst, N + (jnp.arange(npad, dtype=jnp.int32) % NTILES)])
    src3 = src_p.reshape(NTILES, NCHUNK, CHUNK)
    srcx = jnp.stack([src3, src3 + N])
    dst1_3 = jnp.where(dst_p < R1, dst_p,
                       R1 + spread).reshape(NTILES, NCHUNK, CHUNK)
    dst2_3 = jnp.where(dst_p >= R1, dst_p - TAILOFF,
                       spread).reshape(NTILES, NCHUNK, CHUNK)
    tail_edge = (dst_p >= R1).reshape(1, NTILES, NCHUNK, CHUNK)
    fake_src = (jnp.arange(EPAD, dtype=jnp.int32) % 512).reshape(
        1, NTILES, NCHUNK, CHUNK)
    srcx2 = jnp.where(tail_edge, srcx, fake_src)
    zeros_blk = jnp.zeros((NSP1 // NTILES, HH), jnp.float32)
    b_ih2 = b_ih.reshape(1, 3 * H)
    b_hh2 = b_hh.reshape(1, 3 * H)

    h = x
    for i in range(L):
        m2, gh = _tc_a(h, weight[i], w_hh, b_hh2)
        m_flat = m2.reshape(2 * N, HH)
        agg1 = _sc_scatter1(m_flat, srcx, dst1_3, zeros_blk)
        agg2 = _sc_scatter2(m_flat, srcx2, dst2_3, zeros_blk)
        h = _tc_b(agg1, agg2, h, gh, w_ih, b_ih2)
    return h


# speedup vs baseline: 1.1727x; 1.1727x over previous
import functools

import jax
import jax.numpy as jnp
from jax import lax
from jax.experimental import pallas as pl
from jax.experimental.pallas import tpu as pltpu
from jax.experimental.pallas import tpu_sc as plsc

N = 10000
E = 160000
H = 256
HH = 128
L = 2

NTILES = 16
CHUNK = 128
NCHUNK = 80
EPAD = NTILES * NCHUNK * CHUNK

R1 = 9712
NSP1 = 9728
NSP2 = 1024
OUT1 = 10240
TAILOFF = 9000

BR = 1000
TAILBLK = N // BR - 1
TAILROW = R1 - TAILBLK * BR


def _tc_a_body(h_ref, w_ref, whh_ref, bhh_ref, m2_ref, gh_ref):
    h_blk = h_ref[...]
    m = jnp.dot(h_blk, w_ref[...], preferred_element_type=jnp.float32)
    m2_ref[0] = m[:, :HH]
    m2_ref[1] = m[:, HH:]
    gh = lax.dot_general(h_blk, whh_ref[...], (((1,), (1,)), ((), ())),
                         preferred_element_type=jnp.float32)
    gh_ref[...] = gh + bhh_ref[...]


def _tc_a(h, w, w_hh, b_hh2):
    return pl.pallas_call(
        _tc_a_body,
        grid=(N // BR,),
        in_specs=[
            pl.BlockSpec((BR, H), lambda r: (r, 0)),
            pl.BlockSpec((H, H), lambda r: (0, 0)),
            pl.BlockSpec((3 * H, H), lambda r: (0, 0)),
            pl.BlockSpec((1, 3 * H), lambda r: (0, 0)),
        ],
        out_specs=[
            pl.BlockSpec((2, BR, HH), lambda r: (0, r, 0)),
            pl.BlockSpec((BR, 3 * H), lambda r: (r, 0)),
        ],
        out_shape=[
            jax.ShapeDtypeStruct((2, N, HH), jnp.float32),
            jax.ShapeDtypeStruct((N, 3 * H), jnp.float32),
        ],
    )(h, w, w_hh, b_hh2)


def _tc_b_body(agg1_ref, agg2_ref, h_ref, gh_ref, wih_ref, bih_ref, out_ref):
    r = pl.program_id(0)
    row = lax.broadcasted_iota(jnp.int32, (BR, HH), 0)
    tail = jnp.logical_and(r == TAILBLK, row >= TAILROW)
    lo = jnp.where(tail, agg2_ref[0], agg1_ref[0])
    hi = jnp.where(tail, agg2_ref[1], agg1_ref[1])
    agg = jnp.concatenate([lo, hi], axis=1)
    gi = lax.dot_general(agg, wih_ref[...], (((1,), (1,)), ((), ())),
                         preferred_element_type=jnp.float32)
    gi = gi + bih_ref[...]
    gh = gh_ref[...]
    h_blk = h_ref[...]
    i_r = gi[:, :H]
    i_z = gi[:, H:2 * H]
    i_n = gi[:, 2 * H:]
    h_r = gh[:, :H]
    h_z = gh[:, H:2 * H]
    h_n = gh[:, 2 * H:]
    r = jax.nn.sigmoid(i_r + h_r)
    z = jax.nn.sigmoid(i_z + h_z)
    n = jnp.tanh(i_n + r * h_n)
    out_ref[...] = (1.0 - z) * n + z * h_blk


def _tc_b(agg1, agg2, h, gh, w_ih, b_ih2):
    return pl.pallas_call(
        _tc_b_body,
        grid=(N // BR,),
        in_specs=[
            pl.BlockSpec((2, BR, HH), lambda r: (0, r, 0)),
            pl.BlockSpec((2, BR, HH), lambda r: (0, 0, 0)),
            pl.BlockSpec((BR, H), lambda r: (r, 0)),
            pl.BlockSpec((BR, 3 * H), lambda r: (r, 0)),
            pl.BlockSpec((3 * H, H), lambda r: (0, 0)),
            pl.BlockSpec((1, 3 * H), lambda r: (0, 0)),
        ],
        out_specs=pl.BlockSpec((BR, H), lambda r: (r, 0)),
        out_shape=jax.ShapeDtypeStruct((N, H), jnp.float32),
    )(agg1, agg2, h, gh, w_ih, b_ih2)


def _make_sc(nsp, out_rows):
    zrows = nsp // NTILES
    assert zrows % 8 == 0

    def body(m_hbm, srcx_hbm, dst_hbm, zeros_hbm, out_hbm,
             src_v, dst_v, buf0, buf1, agg_sh, sem0, sem1):
        c = lax.axis_index("c")
        s = lax.axis_index("s")
        pltpu.sync_copy(srcx_hbm.at[c, s], src_v)
        pltpu.sync_copy(dst_hbm.at[s], dst_v)
        pltpu.sync_copy(zeros_hbm.at[pl.ds(0, zrows)],
                        agg_sh.at[pl.ds(s * zrows, zrows)])
        plsc.subcore_barrier()

        pltpu.async_copy(m_hbm.at[src_v.at[0]], buf0, sem0)
        pltpu.async_copy(m_hbm.at[src_v.at[1]], buf1, sem1)

        def loop(j2, carry):
            j = j2 * 2
            pltpu.make_async_copy(m_hbm.at[src_v.at[j]], buf0, sem0).wait()
            pltpu.sync_copy(buf0, agg_sh.at[dst_v.at[j]], add=True)

            @pl.when(j2 < NCHUNK // 2 - 1)
            def _():
                pltpu.async_copy(m_hbm.at[src_v.at[j + 2]], buf0, sem0)

            pltpu.make_async_copy(m_hbm.at[src_v.at[j + 1]], buf1,
                                  sem1).wait()
            pltpu.sync_copy(buf1, agg_sh.at[dst_v.at[j + 1]], add=True)

            @pl.when(j2 < NCHUNK // 2 - 1)
            def _():
                pltpu.async_copy(m_hbm.at[src_v.at[j + 3]], buf1, sem1)

            return carry

        lax.fori_loop(0, NCHUNK // 2, loop, 0)
        plsc.subcore_barrier()
        pltpu.sync_copy(agg_sh.at[pl.ds(s * zrows, zrows)],
                        out_hbm.at[c, pl.ds(s * zrows, zrows)])

    return functools.partial(
        pl.kernel,
        out_type=jax.ShapeDtypeStruct((2, out_rows, HH), jnp.float32),
        mesh=plsc.VectorSubcoreMesh(core_axis_name="c", subcore_axis_name="s"),
        scratch_types=(
            [pltpu.VMEM((NCHUNK, CHUNK), jnp.int32) for _ in range(2)]
            + [pltpu.VMEM((CHUNK, HH), jnp.float32) for _ in range(2)]
            + [pltpu.VMEM_SHARED((nsp, HH), jnp.float32)]
            + [pltpu.SemaphoreType.DMA for _ in range(2)]
        ),
    )(body)


_sc_scatter1 = _make_sc(NSP1, OUT1)
_sc_scatter2 = _make_sc(NSP2, NSP2)


@jax.jit
def kernel(x, edge_index, weight, w_ih, w_hh, b_ih, b_hh):
    src = edge_index[0]
    dst = edge_index[1]
    npad = EPAD - E
    spread = jnp.arange(EPAD, dtype=jnp.int32) % NTILES
    src_p = jnp.concatenate(
        [src, jnp.arange(npad, dtype=jnp.int32) % NTILES])
    dst_p = jnp.concatenate(
        [dst, N + (jnp.arange(npad, dtype=jnp.int32) % NTILES)])
    src3 = src_p.reshape(NTILES, NCHUNK, CHUNK)
    srcx = jnp.stack([src3, src3 + N])
    dst1_3 = jnp.where(dst_p < R1, dst_p,
                       R1 + spread).reshape(NTILES, NCHUNK, CHUNK)
    dst2_3 = jnp.where(dst_p >= R1, dst_p - TAILOFF,
                       spread).reshape(NTILES, NCHUNK, CHUNK)
    zeros_blk = jnp.zeros((NSP1 // NTILES, HH), jnp.float32)
    b_ih2 = b_ih.reshape(1, 3 * H)
    b_hh2 = b_hh.reshape(1, 3 * H)

    h = x
    for i in range(L):
        m2, gh = _tc_a(h, weight[i], w_hh, b_hh2)
        m_flat = m2.reshape(2 * N, HH)
        agg1 = _sc_scatter1(m_flat, srcx, dst1_3, zeros_blk)
        agg2 = _sc_scatter2(m_flat, srcx, dst2_3, zeros_blk)
        h = _tc_b(agg1, agg2, h, gh, w_ih, b_ih2)
    return h
